# Initial kernel scaffold; baseline (speedup 1.0000x reference)
import jax
import jax.numpy as jnp
from jax.experimental import pallas as pl


def kernel(X, W9_64, b9_64, W64_64, b64_64, W96_1024, b96_1024, Wc1, bc1, Wc2, bc2, Wc3, bc3, Wc4, bc4):
    raise NotImplementedError("write your pallas kernel here")


# speedup vs baseline: 3.3077x; 3.3077x over previous
import functools

import jax
import jax.numpy as jnp
from jax.experimental import pallas as pl
from jax.experimental.pallas import tpu as pltpu

_K = 20
_ROWS = 256
_TAIL_ROWS = 1024


def _mmT(a, b):
    return jax.lax.dot_general(a, b, (((1,), (1,)), ((), ())),
                               preferred_element_type=jnp.float32)


def _prep_body(f_ref, w_ref, b_ref, yhi_ref, ylo_ref, x2r_ref, x2c_ref):
    f = f_ref[...]
    y = _mmT(f, w_ref[...]) + b_ref[...]
    yhi = y.astype(jnp.bfloat16)
    ylo = (y - yhi.astype(jnp.float32)).astype(jnp.bfloat16)
    yhi_ref[...] = yhi
    ylo_ref[...] = ylo
    f2 = f * f
    x2c_ref[...] = jnp.sum(f2, axis=1, keepdims=True)
    x2r_ref[...] = _mmT(jnp.ones((1, f.shape[1]), jnp.float32), f2)


def _knn_body(nk, fblk_ref, ffull_ref, x2r_ref, x2c_ref, yhi_ref, ylo_ref,
              out_ref):
    rows = fblk_ref.shape[0]
    n = ffull_ref.shape[0]
    g = _mmT(fblk_ref[...], ffull_ref[...])
    scores = (x2c_ref[...] + x2r_ref[...]) - 2.0 * g
    iota = jax.lax.broadcasted_iota(jnp.int32, (rows, n), 1)
    yhi = yhi_ref[...]
    ylo = ylo_ref[...]
    acc = jnp.full((rows, yhi.shape[1]), -jnp.inf, dtype=jnp.float32)
    for _ in range(nk):
        m = jnp.min(scores, axis=1, keepdims=True)
        is_min = scores <= m
        idx = jnp.min(jnp.where(is_min, iota, n), axis=1, keepdims=True)
        onehot = iota == idx
        oh = onehot.astype(jnp.bfloat16)
        ghi = jax.lax.dot_general(oh, yhi, (((1,), (0,)), ((), ())),
                                  preferred_element_type=jnp.float32)
        glo = jax.lax.dot_general(oh, ylo, (((1,), (0,)), ((), ())),
                                  preferred_element_type=jnp.float32)
        acc = jnp.maximum(acc, ghi + glo)
        scores = jnp.where(onehot, jnp.inf, scores)
    out_ref[...] = acc


def _knn_layer(f, w, b, nk=None, rows=None):
    nk = _K if nk is None else nk
    rows = _ROWS if rows is None else rows
    n, d = f.shape
    dd = w.shape[0]
    yhi, ylo, x2r, x2c = pl.pallas_call(
        _prep_body,
        out_shape=(
            jax.ShapeDtypeStruct((n, dd), jnp.bfloat16),
            jax.ShapeDtypeStruct((n, dd), jnp.bfloat16),
            jax.ShapeDtypeStruct((1, n), jnp.float32),
            jax.ShapeDtypeStruct((n, 1), jnp.float32),
        ),
    )(f, w, b)
    out = pl.pallas_call(
        functools.partial(_knn_body, nk),
        grid=(n // rows,),
        in_specs=[
            pl.BlockSpec((rows, d), lambda i: (i, 0)),
            pl.BlockSpec((n, d), lambda i: (0, 0)),
            pl.BlockSpec((1, n), lambda i: (0, 0)),
            pl.BlockSpec((rows, 1), lambda i: (i, 0)),
            pl.BlockSpec((n, dd), lambda i: (0, 0)),
            pl.BlockSpec((n, dd), lambda i: (0, 0)),
        ],
        out_specs=pl.BlockSpec((rows, dd), lambda i: (i, 0)),
        out_shape=jax.ShapeDtypeStruct((n, dd), jnp.float32),
        compiler_params=pltpu.CompilerParams(
            dimension_semantics=("arbitrary",),
            vmem_limit_bytes=128 * 1024 * 1024,
        ),
    )(f, f, x2r, x2c, yhi, ylo)
    return out


def _tail_body(x1_ref, x2_ref, x3_ref, w96_ref, b96_ref, a1_ref, a2_ref,
               a3_ref, a4_ref, bc1_ref, wc2_ref, bc2_ref, wc3_ref, bc3_ref,
               wc4_ref, bc4_ref, out_ref):
    x1 = x1_ref[...]
    x2 = x2_ref[...]
    x3 = x3_ref[...]
    ii = jax.lax.broadcasted_iota(jnp.int32, (64, 32), 0)
    jj = jax.lax.broadcasted_iota(jnp.int32, (64, 32), 1)
    ee = (ii == 2 * jj).astype(jnp.float32)
    eo = (ii == 2 * jj + 1).astype(jnp.float32)

    def pool(a):
        pe = jax.lax.dot_general(a, ee, (((1,), (0,)), ((), ())),
                                 preferred_element_type=jnp.float32)
        po = jax.lax.dot_general(a, eo, (((1,), (0,)), ((), ())),
                                 preferred_element_type=jnp.float32)
        return jnp.maximum(pe, po)

    xp = jnp.concatenate([pool(x1), pool(x2), pool(x3)], axis=1)
    xf = _mmT(xp, w96_ref[...]) + b96_ref[...]
    h = (_mmT(x1, a1_ref[...]) + _mmT(x2, a2_ref[...]) +
         _mmT(x3, a3_ref[...]) + _mmT(xf, a4_ref[...]) + bc1_ref[...])
    h = jnp.maximum(h, 0.0)
    h = jnp.maximum(_mmT(h, wc2_ref[...]) + bc2_ref[...], 0.0)
    h = jnp.maximum(_mmT(h, wc3_ref[...]) + bc3_ref[...], 0.0)
    out_ref[...] = _mmT(h, wc4_ref[...]) + bc4_ref[...]


def _tail(x1, x2, x3, w96, b96, wc1, bc1, wc2, bc2, wc3, bc3, wc4p, bc4p,
          rows=None):
    rows = _TAIL_ROWS if rows is None else rows
    n = x1.shape[0]
    blk = lambda r, c: pl.BlockSpec((r, c), lambda i: (i, 0))
    full = lambda shape: pl.BlockSpec(shape, lambda i: (0, 0))
    a1, a2, a3, a4 = (wc1[:, :64], wc1[:, 64:128], wc1[:, 128:192],
                      wc1[:, 192:])
    return pl.pallas_call(
        _tail_body,
        grid=(n // rows,),
        in_specs=[
            blk(rows, 64), blk(rows, 64), blk(rows, 64),
            full(w96.shape), full(b96.shape),
            full(a1.shape), full(a2.shape), full(a3.shape), full(a4.shape),
            full(bc1.shape), full(wc2.shape), full(bc2.shape),
            full(wc3.shape), full(bc3.shape), full(wc4p.shape),
            full(bc4p.shape),
        ],
        out_specs=pl.BlockSpec((rows, 128), lambda i: (i, 0)),
        out_shape=jax.ShapeDtypeStruct((n, 128), jnp.float32),
        compiler_params=pltpu.CompilerParams(
            dimension_semantics=("arbitrary",),
            vmem_limit_bytes=128 * 1024 * 1024,
        ),
    )(x1, x2, x3, w96, b96, a1, a2, a3, a4, bc1, wc2, bc2, wc3, bc3, wc4p,
      bc4p)


def kernel(X, W9_64, b9_64, W64_64, b64_64, W96_1024, b96_1024,
           Wc1, bc1, Wc2, bc2, Wc3, bc3, Wc4, bc4):
    xp = jnp.pad(X, ((0, 0), (0, 7)))
    w9p = jnp.pad(W9_64, ((0, 0), (0, 7)))
    x1 = _knn_layer(xp, w9p, b9_64.reshape(1, -1))
    x2 = _knn_layer(x1, W64_64, b64_64.reshape(1, -1))
    x3 = _knn_layer(x2, W64_64, b64_64.reshape(1, -1))
    wc4p = jnp.pad(Wc4, ((0, 125), (0, 0)))
    bc4p = jnp.pad(bc4, (0, 125)).reshape(1, -1)
    out = _tail(x1, x2, x3, W96_1024, b96_1024.reshape(1, -1),
                Wc1, bc1.reshape(1, -1), Wc2, bc2.reshape(1, -1),
                Wc3, bc3.reshape(1, -1), wc4p, bc4p)
    return out[:, :3]


# speedup vs baseline: 5.7918x; 1.7510x over previous
import functools

import jax
import jax.numpy as jnp
from jax import lax
from jax.experimental import pallas as pl
from jax.experimental.pallas import tpu as pltpu
from jax.experimental.pallas import tpu_sc as plsc

_K = 20
_KPAD = 24
_ROWS = 256
_TAIL_ROWS = 1024
_SC_NC = 2
_SC_NS = 16
_NW = _SC_NC * _SC_NS


def _mmT(a, b):
    return jax.lax.dot_general(a, b, (((1,), (1,)), ((), ())),
                               preferred_element_type=jnp.float32)


def _prep_body(f_ref, w_ref, b_ref, y_ref, x2r_ref, x2c_ref):
    f = f_ref[...]
    y_ref[...] = _mmT(f, w_ref[...]) + b_ref[...]
    f2 = f * f
    x2c_ref[...] = jnp.sum(f2, axis=1, keepdims=True)
    x2r_ref[...] = _mmT(jnp.ones((1, f.shape[1]), jnp.float32), f2)


def _knn_body(nk, dused, fblk_ref, ffull_ref, x2r_ref, x2c_ref, idx_ref):
    rows = fblk_ref.shape[0]
    n = ffull_ref.shape[0]
    g = _mmT(fblk_ref[:, :dused], ffull_ref[:, :dused])
    scores = (x2c_ref[...] + x2r_ref[...]) - 2.0 * g
    iota = jax.lax.broadcasted_iota(jnp.int32, (rows, n), 1)
    picked = []
    for _ in range(nk):
        m = jnp.min(scores, axis=1, keepdims=True)
        is_min = scores <= m
        idx = jnp.min(jnp.where(is_min, iota, n), axis=1, keepdims=True)
        picked.append(idx)
        scores = jnp.where(iota == idx, jnp.inf, scores)
    picked += [picked[0]] * (_KPAD - nk)
    idx_ref[...] = jnp.concatenate(picked, axis=1)


def _knn_indices(f, x2r, x2c, dused, nk=None, rows=None):
    nk = _K if nk is None else nk
    rows = _ROWS if rows is None else rows
    n, d = f.shape
    return pl.pallas_call(
        functools.partial(_knn_body, nk, dused),
        grid=(n // rows,),
        in_specs=[
            pl.BlockSpec((rows, d), lambda i: (i, 0)),
            pl.BlockSpec((n, d), lambda i: (0, 0)),
            pl.BlockSpec((1, n), lambda i: (0, 0)),
            pl.BlockSpec((rows, 1), lambda i: (i, 0)),
        ],
        out_specs=pl.BlockSpec((rows, _KPAD), lambda i: (i, 0)),
        out_shape=jax.ShapeDtypeStruct((n, _KPAD), jnp.int32),
        compiler_params=pltpu.CompilerParams(
            dimension_semantics=("arbitrary",),
            vmem_limit_bytes=128 * 1024 * 1024,
        ),
    )(f, f, x2r, x2c)


def _gather_max(y, idx_flat, ch=8):
    n, d = y.shape
    rows_w = n // _NW
    nch = rows_w // ch
    nd = d // 16
    mesh = plsc.VectorSubcoreMesh(core_axis_name="c", subcore_axis_name="s",
                                  num_cores=_SC_NC, num_subcores=_SC_NS)

    @functools.partial(
        pl.kernel, mesh=mesh,
        out_type=jax.ShapeDtypeStruct((n, d), jnp.float32),
        scratch_types=[
            pltpu.VMEM((rows_w * _KPAD,), jnp.int32),
            pltpu.VMEM((ch * _KPAD, d), jnp.float32),
            pltpu.VMEM((ch * _KPAD, d), jnp.float32),
            pltpu.VMEM((rows_w, d), jnp.float32),
            pltpu.SemaphoreType.DMA,
            pltpu.SemaphoreType.DMA,
        ],
    )
    def k(y_hbm, idxf_hbm, out_hbm, idx_v, buf0, buf1, out_v, sem0, sem1):
        wid = lax.axis_index("s") * _SC_NC + lax.axis_index("c")
        base = wid * rows_w
        pltpu.sync_copy(idxf_hbm.at[pl.ds(base * _KPAD, rows_w * _KPAD)],
                        idx_v)
        bufs = (buf0, buf1)
        sems = (sem0, sem1)
        for b in range(2):
            pltpu.async_copy(
                y_hbm.at[idx_v.at[pl.ds(b * ch * _KPAD, ch * _KPAD)]],
                bufs[b], sems[b])

        @pl.loop(0, nch, step=2)
        def _pair(g):
            for b in range(2):
                c = g + b
                buf = bufs[b]
                pltpu.make_async_copy(
                    y_hbm.at[idx_v.at[pl.ds(0, ch * _KPAD)]],
                    buf, sems[b]).wait()

                @pl.loop(0, ch)
                def _row(r):
                    for j in range(nd):
                        acc = buf[r * _KPAD, pl.ds(j * 16, 16)]
                        for t in range(1, _KPAD):
                            acc = jnp.maximum(
                                acc, buf[r * _KPAD + t, pl.ds(j * 16, 16)])
                        out_v[c * ch + r, pl.ds(j * 16, 16)] = acc

                @pl.when(c + 2 < nch)
                def _():
                    pltpu.async_copy(
                        y_hbm.at[idx_v.at[pl.ds((c + 2) * ch * _KPAD,
                                                ch * _KPAD)]],
                        buf, sems[b])

        pltpu.sync_copy(out_v, out_hbm.at[pl.ds(base, rows_w)])

    return k(y, idx_flat)


def _knn_layer(f, w, b, dused, nk=None, rows=None):
    n, d = f.shape
    dd = w.shape[0]
    y, x2r, x2c = pl.pallas_call(
        _prep_body,
        out_shape=(
            jax.ShapeDtypeStruct((n, dd), jnp.float32),
            jax.ShapeDtypeStruct((1, n), jnp.float32),
            jax.ShapeDtypeStruct((n, 1), jnp.float32),
        ),
    )(f, w, b)
    idx = _knn_indices(f, x2r, x2c, dused, nk=nk, rows=rows)
    return _gather_max(y, idx.reshape(n * _KPAD))


def _tail_body(x1_ref, x2_ref, x3_ref, w96_ref, b96_ref, a1_ref, a2_ref,
               a3_ref, a4_ref, bc1_ref, wc2_ref, bc2_ref, wc3_ref, bc3_ref,
               wc4_ref, bc4_ref, out_ref):
    x1 = x1_ref[...]
    x2 = x2_ref[...]
    x3 = x3_ref[...]
    ii = jax.lax.broadcasted_iota(jnp.int32, (128, 32), 0)
    jj = jax.lax.broadcasted_iota(jnp.int32, (128, 32), 1)
    ee = (ii == 2 * jj).astype(jnp.float32)
    eo = (ii == 2 * jj + 1).astype(jnp.float32)

    def pool(a):
        pe = jax.lax.dot_general(a, ee, (((1,), (0,)), ((), ())),
                                 preferred_element_type=jnp.float32)
        po = jax.lax.dot_general(a, eo, (((1,), (0,)), ((), ())),
                                 preferred_element_type=jnp.float32)
        return jnp.maximum(pe, po)

    xp = jnp.concatenate([pool(x1), pool(x2), pool(x3)], axis=1)
    xf = _mmT(xp, w96_ref[...]) + b96_ref[...]
    h = (_mmT(x1, a1_ref[...]) + _mmT(x2, a2_ref[...]) +
         _mmT(x3, a3_ref[...]) + _mmT(xf, a4_ref[...]) + bc1_ref[...])
    h = jnp.maximum(h, 0.0)
    h = jnp.maximum(_mmT(h, wc2_ref[...]) + bc2_ref[...], 0.0)
    h = jnp.maximum(_mmT(h, wc3_ref[...]) + bc3_ref[...], 0.0)
    out_ref[...] = _mmT(h, wc4_ref[...]) + bc4_ref[...]


def _tail(x1, x2, x3, w96, b96, wc1, bc1, wc2, bc2, wc3, bc3, wc4p, bc4p,
          rows=None):
    rows = _TAIL_ROWS if rows is None else rows
    n = x1.shape[0]
    blk = lambda r, c: pl.BlockSpec((r, c), lambda i: (i, 0))
    full = lambda shape: pl.BlockSpec(shape, lambda i: (0, 0))
    zp = lambda a: jnp.pad(a, ((0, 0), (0, 64)))
    a1, a2, a3 = (zp(wc1[:, :64]), zp(wc1[:, 64:128]), zp(wc1[:, 128:192]))
    a4 = wc1[:, 192:]
    return pl.pallas_call(
        _tail_body,
        grid=(n // rows,),
        in_specs=[
            blk(rows, 128), blk(rows, 128), blk(rows, 128),
            full(w96.shape), full(b96.shape),
            full(a1.shape), full(a2.shape), full(a3.shape), full(a4.shape),
            full(bc1.shape), full(wc2.shape), full(bc2.shape),
            full(wc3.shape), full(bc3.shape), full(wc4p.shape),
            full(bc4p.shape),
        ],
        out_specs=pl.BlockSpec((rows, 128), lambda i: (i, 0)),
        out_shape=jax.ShapeDtypeStruct((n, 128), jnp.float32),
        compiler_params=pltpu.CompilerParams(
            dimension_semantics=("arbitrary",),
            vmem_limit_bytes=128 * 1024 * 1024,
        ),
    )(x1, x2, x3, w96, b96, a1, a2, a3, a4, bc1, wc2, bc2, wc3, bc3, wc4p,
      bc4p)


def kernel(X, W9_64, b9_64, W64_64, b64_64, W96_1024, b96_1024,
           Wc1, bc1, Wc2, bc2, Wc3, bc3, Wc4, bc4):
    xp = jnp.pad(X, ((0, 0), (0, 7)))
    w9p = jnp.pad(W9_64, ((0, 64), (0, 7)))
    b9p = jnp.pad(b9_64, (0, 64)).reshape(1, -1)
    w64p = jnp.pad(W64_64, ((0, 64), (0, 64)))
    b64p = jnp.pad(b64_64, (0, 64)).reshape(1, -1)
    x1 = _knn_layer(xp, w9p, b9p, dused=16)
    x2 = _knn_layer(x1, w64p, b64p, dused=64)
    x3 = _knn_layer(x2, w64p, b64p, dused=64)
    wc4p = jnp.pad(Wc4, ((0, 125), (0, 0)))
    bc4p = jnp.pad(bc4, (0, 125)).reshape(1, -1)
    out = _tail(x1, x2, x3, W96_1024, b96_1024.reshape(1, -1),
                Wc1, bc1.reshape(1, -1), Wc2, bc2.reshape(1, -1),
                Wc3, bc3.reshape(1, -1), wc4p, bc4p)
    return out[:, :3]
